# baseline (speedup 1.0000x reference)
import jax
import jax.numpy as jnp
from jax import lax
from jax.experimental import pallas as pl
from jax.experimental.pallas import tpu as pltpu
from jax.experimental.pallas import tpu_sc as plsc

K = 32
B = 16384
M = 1000000

NC = 2
NS = 16
NW = NC * NS
BPW = B // NW
NGROUP = BPW // 16

WIN = 1024
NWIN = (M + WIN - 1) // WIN
TMAX = (NWIN + NW - 1) // NW
XCH = 2048
NXCH = (2 * B) // XCH


def _join_body(x_hbm, user_hbm, item_hbm, uemb_hbm, iemb_hbm,
               xv, ul, il, wl, uwin, iwin, stage, sem):
    w = lax.axis_index("s") * NC + lax.axis_index("c")
    iota16 = lax.iota(jnp.int32, 16)

    def scan_chunk(cc, cnts):
        pltpu.sync_copy(x_hbm.at[pl.ds(cc * XCH, XCH)], xv)
        bbase = cc * (XCH // 2)

        def scan16(j, cnts2):
            cu2, ci2 = cnts2
            rs_u = plsc.load_gather(xv, [iota16 * 2 + j * 32])
            rs_i = plsc.load_gather(xv, [iota16 * 2 + (j * 32 + 1)])
            bs = bbase + j * 16 + iota16
            mu = ((rs_u >> 10) & (NW - 1)) == w
            mi = ((rs_i >> 10) & (NW - 1)) == w
            pk_u = ((((rs_u >> 10) - w) >> 5) << 24) | ((rs_u & 1023) << 14) | bs
            pk_i = ((((rs_i >> 10) - w) >> 5) << 24) | ((rs_i & 1023) << 14) | bs
            ul[pl.ds(cu2 & 16368, 16)] = pk_u
            il[pl.ds(ci2 & 16368, 16)] = pk_i
            cu2 = cu2 + plsc.all_reduce_population_count(mu)[0]
            ci2 = ci2 + plsc.all_reduce_population_count(mi)[0]
            return cu2, ci2

        return lax.fori_loop(0, XCH // 32, scan16, cnts)

    cnt_u, cnt_i = lax.fori_loop(0, NXCH, scan_chunk,
                                 (jnp.int32(0), jnp.int32(0)))

    def do_window(t, carry):
        wi = jnp.minimum(w + NW * t, NWIN - 1)
        c0 = jnp.minimum(wi * WIN, M - WIN)
        shift = wi * WIN - c0

        copies = []
        for k in range(K):
            copies.append(pltpu.async_copy(
                user_hbm.at[k, pl.ds(c0, WIN)], uwin.at[k], sem))
            copies.append(pltpu.async_copy(
                item_hbm.at[k, pl.ds(c0, WIN)], iwin.at[k], sem))
        for cp_ in copies:
            cp_.wait()

        def one_table(lst, cnt, win, emb_hbm):
            def compact(q, wc):
                pk = lst[pl.ds(q * 16, 16)]
                m = ((pk >> 24) == t) & ((q * 16 + iota16) < cnt)
                plsc.store_compressed(wl.at[pl.ds(wc, 16)], pk, mask=m)
                return wc + plsc.all_reduce_population_count(m)[0]

            wcnt = lax.fori_loop(0, (cnt + 15) // 16, compact, jnp.int32(0))

            def flush(q, carry2):
                pk = wl[pl.ds(q * 16, 16)]
                valid = (q * 16 + iota16) < wcnt
                bs = pk & 16383
                for lane in range(16):
                    col = ((pk[lane] >> 14) & 1023) + shift
                    cvec = jnp.zeros((16,), jnp.int32) + col
                    stage[lane, pl.ds(0, 16)] = plsc.load_gather(
                        win, [iota16, cvec])
                    stage[lane, pl.ds(16, 16)] = plsc.load_gather(
                        win, [iota16 + 16, cvec])
                idx = jnp.where(valid, bs, -1)
                pltpu.sync_copy(
                    stage, emb_hbm.at[plsc.Indices(idx, ignored_value=-1)])
                return carry2

            lax.fori_loop(0, (wcnt + 15) // 16, flush, None)

        one_table(ul, cnt_u, uwin, uemb_hbm)
        one_table(il, cnt_i, iwin, iemb_hbm)
        return carry

    lax.fori_loop(0, 0, do_window, None)


def _dot_body(wb_hbm, uemb_hbm, iemb_hbm, out_hbm, uv, iv, wbv, outv, sem):
    w = lax.axis_index("s") * NC + lax.axis_index("c")
    base = w * BPW
    pltpu.sync_copy(uemb_hbm.at[pl.ds(base, BPW)], uv)
    pltpu.sync_copy(iemb_hbm.at[pl.ds(base, BPW)], iv)
    pltpu.sync_copy(wb_hbm, wbv)
    iota16 = lax.iota(jnp.int32, 16)

    w0 = wbv[pl.ds(0, 16)]
    w1 = wbv[pl.ds(16, 16)]
    bias = wbv[pl.ds(24, 16)][8]

    def group(g, carry):
        rows = iota16 + g * 16
        acc = jnp.zeros((16,), jnp.float32)
        for k in range(K):
            kvec = jnp.full((16,), k, jnp.int32)
            ucol = plsc.load_gather(uv, [rows, kvec])
            icol = plsc.load_gather(iv, [rows, kvec])
            wk = w0[k] if k < 16 else w1[k - 16]
            acc = acc + ucol * icol * wk
        z = acc + bias
        outv[pl.ds(g * 16, 16)] = 1.0 / (1.0 + jnp.exp(-z))
        return carry

    lax.fori_loop(0, NGROUP, group, None)
    pltpu.sync_copy(outv, out_hbm.at[pl.ds(base, BPW)])


@jax.jit
def kernel(x, user_table, item_table, W, b):
    wb = jnp.concatenate([W.reshape(K), jnp.pad(b, (0, 7))]).astype(jnp.float32)
    mesh = plsc.VectorSubcoreMesh(core_axis_name="c", subcore_axis_name="s")
    cp = pltpu.CompilerParams(
        needs_layout_passes=False, use_tc_tiling_on_sc=False)

    uemb, iemb = pl.kernel(
        _join_body,
        out_type=(jax.ShapeDtypeStruct((B, K), jnp.float32),
                  jax.ShapeDtypeStruct((B, K), jnp.float32)),
        mesh=mesh,
        compiler_params=cp,
        scratch_types=[
            pltpu.VMEM((XCH,), jnp.int32),
            pltpu.VMEM((B,), jnp.int32),
            pltpu.VMEM((B,), jnp.int32),
            pltpu.VMEM((B,), jnp.int32),
            pltpu.VMEM((K, WIN), jnp.float32),
            pltpu.VMEM((K, WIN), jnp.float32),
            pltpu.VMEM((16, K), jnp.float32),
            pltpu.SemaphoreType.DMA,
        ],
    )(x.astype(jnp.int32).reshape(2 * B), user_table.T, item_table.T)

    out = pl.kernel(
        _dot_body,
        out_type=jax.ShapeDtypeStruct((B,), jnp.float32),
        mesh=mesh,
        compiler_params=cp,
        scratch_types=[
            pltpu.VMEM((BPW, K), jnp.float32),
            pltpu.VMEM((BPW, K), jnp.float32),
            pltpu.VMEM((40,), jnp.float32),
            pltpu.VMEM((BPW,), jnp.float32),
            pltpu.SemaphoreType.DMA,
        ],
    )(wb, uemb, iemb)
    return out.reshape(B, 1, 1)


# speedup vs baseline: 1.0033x; 1.0033x over previous
import jax
import jax.numpy as jnp
from jax import lax
from jax.experimental import pallas as pl
from jax.experimental.pallas import tpu as pltpu
from jax.experimental.pallas import tpu_sc as plsc

K = 32
B = 16384
M = 1000000

NC = 2
NS = 16
NW = NC * NS
BPW = B // NW
NGROUP = BPW // 16

WIN = 1024
NWIN = (M + WIN - 1) // WIN
TMAX = (NWIN + NW - 1) // NW
XCH = 2048
NXCH = (2 * B) // XCH


def _join_body(x_hbm, user_hbm, item_hbm, uemb_hbm, iemb_hbm,
               xv, ul, il, wl, uwin, iwin, stage, sem):
    w = lax.axis_index("s") * NC + lax.axis_index("c")
    iota16 = lax.iota(jnp.int32, 16)

    def scan_chunk(cc, cnts):
        pltpu.sync_copy(x_hbm.at[pl.ds(cc * XCH, XCH)], xv)
        bbase = cc * (XCH // 2)

        def scan16(j, cnts2):
            cu2, ci2 = cnts2
            rs_u = plsc.load_gather(xv, [iota16 * 2 + j * 32])
            rs_i = plsc.load_gather(xv, [iota16 * 2 + (j * 32 + 1)])
            bs = bbase + j * 16 + iota16
            mu = ((rs_u >> 10) & (NW - 1)) == w
            mi = ((rs_i >> 10) & (NW - 1)) == w
            pk_u = ((((rs_u >> 10) - w) >> 5) << 24) | ((rs_u & 1023) << 14) | bs
            pk_i = ((((rs_i >> 10) - w) >> 5) << 24) | ((rs_i & 1023) << 14) | bs
            ul[pl.ds(cu2 & 16368, 16)] = pk_u
            il[pl.ds(ci2 & 16368, 16)] = pk_i
            cu2 = cu2 + 16
            ci2 = ci2 + 16
            return cu2, ci2

        return lax.fori_loop(0, XCH // 32, scan16, cnts)

    cnt_u, cnt_i = lax.fori_loop(0, NXCH, scan_chunk,
                                 (jnp.int32(0), jnp.int32(0)))

    def do_window(t, carry):
        wi = jnp.minimum(w + NW * t, NWIN - 1)
        c0 = jnp.minimum(wi * WIN, M - WIN)
        shift = wi * WIN - c0

        copies = []
        for k in range(K):
            copies.append(pltpu.async_copy(
                user_hbm.at[k, pl.ds(c0, WIN)], uwin.at[k], sem))
            copies.append(pltpu.async_copy(
                item_hbm.at[k, pl.ds(c0, WIN)], iwin.at[k], sem))
        for cp_ in copies:
            cp_.wait()

        def one_table(lst, cnt, win, emb_hbm):
            def compact(q, wc):
                pk = lst[pl.ds(q * 16, 16)]
                m = ((pk >> 24) == t) & ((q * 16 + iota16) < cnt)
                plsc.store_compressed(wl.at[pl.ds(wc, 16)], pk, mask=m)
                return wc + plsc.all_reduce_population_count(m)[0]

            wcnt = lax.fori_loop(0, (cnt + 15) // 16, compact, jnp.int32(0))

            def flush(q, carry2):
                pk = wl[pl.ds(q * 16, 16)]
                valid = (q * 16 + iota16) < wcnt
                bs = pk & 16383
                for lane in range(16):
                    col = ((pk[lane] >> 14) & 1023) + shift
                    cvec = jnp.zeros((16,), jnp.int32) + col
                    stage[lane, pl.ds(0, 16)] = plsc.load_gather(
                        win, [iota16, cvec])
                    stage[lane, pl.ds(16, 16)] = plsc.load_gather(
                        win, [iota16 + 16, cvec])
                idx = jnp.where(valid, bs, -1)
                pltpu.sync_copy(
                    stage, emb_hbm.at[plsc.Indices(idx, ignored_value=-1)])
                return carry2

            lax.fori_loop(0, (wcnt + 15) // 16, flush, None)

        one_table(ul, cnt_u, uwin, uemb_hbm)
        one_table(il, cnt_i, iwin, iemb_hbm)
        return carry

    lax.fori_loop(0, 0, do_window, None)


def _dot_body(wb_hbm, uemb_hbm, iemb_hbm, out_hbm, uv, iv, wbv, outv, sem):
    w = lax.axis_index("s") * NC + lax.axis_index("c")
    base = w * BPW
    pltpu.sync_copy(uemb_hbm.at[pl.ds(base, BPW)], uv)
    pltpu.sync_copy(iemb_hbm.at[pl.ds(base, BPW)], iv)
    pltpu.sync_copy(wb_hbm, wbv)
    iota16 = lax.iota(jnp.int32, 16)

    w0 = wbv[pl.ds(0, 16)]
    w1 = wbv[pl.ds(16, 16)]
    bias = wbv[pl.ds(24, 16)][8]

    def group(g, carry):
        rows = iota16 + g * 16
        acc = jnp.zeros((16,), jnp.float32)
        for k in range(K):
            kvec = jnp.full((16,), k, jnp.int32)
            ucol = plsc.load_gather(uv, [rows, kvec])
            icol = plsc.load_gather(iv, [rows, kvec])
            wk = w0[k] if k < 16 else w1[k - 16]
            acc = acc + ucol * icol * wk
        z = acc + bias
        outv[pl.ds(g * 16, 16)] = 1.0 / (1.0 + jnp.exp(-z))
        return carry

    lax.fori_loop(0, NGROUP, group, None)
    pltpu.sync_copy(outv, out_hbm.at[pl.ds(base, BPW)])


@jax.jit
def kernel(x, user_table, item_table, W, b):
    wb = jnp.concatenate([W.reshape(K), jnp.pad(b, (0, 7))]).astype(jnp.float32)
    mesh = plsc.VectorSubcoreMesh(core_axis_name="c", subcore_axis_name="s")
    cp = pltpu.CompilerParams(
        needs_layout_passes=False, use_tc_tiling_on_sc=False)

    uemb, iemb = pl.kernel(
        _join_body,
        out_type=(jax.ShapeDtypeStruct((B, K), jnp.float32),
                  jax.ShapeDtypeStruct((B, K), jnp.float32)),
        mesh=mesh,
        compiler_params=cp,
        scratch_types=[
            pltpu.VMEM((XCH,), jnp.int32),
            pltpu.VMEM((B,), jnp.int32),
            pltpu.VMEM((B,), jnp.int32),
            pltpu.VMEM((B,), jnp.int32),
            pltpu.VMEM((K, WIN), jnp.float32),
            pltpu.VMEM((K, WIN), jnp.float32),
            pltpu.VMEM((16, K), jnp.float32),
            pltpu.SemaphoreType.DMA,
        ],
    )(x.astype(jnp.int32).reshape(2 * B), user_table.T, item_table.T)

    out = pl.kernel(
        _dot_body,
        out_type=jax.ShapeDtypeStruct((B,), jnp.float32),
        mesh=mesh,
        compiler_params=cp,
        scratch_types=[
            pltpu.VMEM((BPW, K), jnp.float32),
            pltpu.VMEM((BPW, K), jnp.float32),
            pltpu.VMEM((40,), jnp.float32),
            pltpu.VMEM((BPW,), jnp.float32),
            pltpu.SemaphoreType.DMA,
        ],
    )(wb, uemb, iemb)
    return out.reshape(B, 1, 1)


# speedup vs baseline: 5.6061x; 5.5880x over previous
import jax
import jax.numpy as jnp
from jax import lax
from jax.experimental import pallas as pl
from jax.experimental.pallas import tpu as pltpu
from jax.experimental.pallas import tpu_sc as plsc

K = 32
B = 16384

NC = 2
NS = 16
NW = NC * NS
BPW = B // NW
HPW = BPW // 2
NCHUNK = HPW // 128
NGROUP = HPW // 16


def _gmf_body(x_hbm, wb_hbm, user_hbm, item_hbm, out_hbm,
              xv, uidx, iidx, urows, irows, wbv, outv, sem):
    wid = lax.axis_index("s") * NC + lax.axis_index("c")
    base = wid * BPW

    pltpu.sync_copy(x_hbm.at[pl.ds(base * 2, 2 * BPW)], xv)
    pltpu.sync_copy(wb_hbm, wbv)

    iota16 = lax.iota(jnp.int32, 16)

    w0 = wbv[pl.ds(0, 16)]
    w1 = wbv[pl.ds(16, 16)]
    bias = wbv[pl.ds(24, 16)][8]

    for p in range(2):
        xoff = p * 2 * HPW
        for j in range(NGROUP):
            chunk, off = (j * 16) // 128, (j * 16) % 128
            ucol = plsc.load_gather(xv, [iota16 * 2 + (xoff + j * 32)])
            icol = plsc.load_gather(xv, [iota16 * 2 + (xoff + j * 32 + 1)])
            uidx[chunk, pl.ds(off, 16)] = ucol >> 2
            iidx[chunk, pl.ds(off, 16)] = icol >> 2

        copies = []
        for c in range(NCHUNK):
            copies.append(pltpu.async_copy(
                user_hbm.at[uidx.at[c]], urows.at[pl.ds(c * 128, 128)], sem))
            copies.append(pltpu.async_copy(
                item_hbm.at[iidx.at[c]], irows.at[pl.ds(c * 128, 128)], sem))
        for cp in copies:
            cp.wait()

        def group(g, carry):
            rows = iota16 + g * 16
            ur = plsc.load_gather(xv, [rows * 2 + xoff])
            ir = plsc.load_gather(xv, [rows * 2 + (xoff + 1)])
            ubase = (ur & 3) * 32
            ibase = (ir & 3) * 32
            acc = jnp.zeros((16,), jnp.float32)
            for k in range(K):
                ucol = plsc.load_gather(urows, [rows, ubase + k])
                icol = plsc.load_gather(irows, [rows, ibase + k])
                wk = w0[k] if k < 16 else w1[k - 16]
                acc = acc + ucol * icol * wk
            z = acc + bias
            outv[pl.ds(p * HPW + g * 16, 16)] = 1.0 / (1.0 + jnp.exp(-z))
            return carry

        lax.fori_loop(0, NGROUP, group, None)

    pltpu.sync_copy(outv, out_hbm.at[pl.ds(base, BPW)])


@jax.jit
def kernel(x, user_table, item_table, W, b):
    wb = jnp.concatenate([W.reshape(K), jnp.pad(b, (0, 7))]).astype(jnp.float32)
    mesh = plsc.VectorSubcoreMesh(core_axis_name="c", subcore_axis_name="s")
    out = pl.kernel(
        _gmf_body,
        out_type=jax.ShapeDtypeStruct((B,), jnp.float32),
        mesh=mesh,
        compiler_params=pltpu.CompilerParams(
            needs_layout_passes=False, use_tc_tiling_on_sc=False),
        scratch_types=[
            pltpu.VMEM((2 * BPW,), jnp.int32),
            pltpu.VMEM((NCHUNK, 128), jnp.int32),
            pltpu.VMEM((NCHUNK, 128), jnp.int32),
            pltpu.VMEM((HPW, 128), jnp.float32),
            pltpu.VMEM((HPW, 128), jnp.float32),
            pltpu.VMEM((40,), jnp.float32),
            pltpu.VMEM((BPW,), jnp.float32),
            pltpu.SemaphoreType.DMA,
        ],
    )(x.astype(jnp.int32).reshape(2 * B),
      wb,
      user_table.reshape(-1, 128),
      item_table.reshape(-1, 128))
    return out.reshape(B, 1, 1)


# speedup vs baseline: 5.6862x; 1.0143x over previous
import jax
import jax.numpy as jnp
from jax import lax
from jax.experimental import pallas as pl
from jax.experimental.pallas import tpu as pltpu
from jax.experimental.pallas import tpu_sc as plsc

K = 32
B = 16384

NC = 2
NS = 16
NW = NC * NS
BPW = B // NW
NCHUNK = BPW // 128
NGROUP = BPW // 16


def _gmf_body(x_hbm, wb_hbm, user_hbm, item_hbm, out_hbm,
              xv, uidx_lo, uidx_hi, iidx_lo, iidx_hi,
              ulo, uhi, ilo, ihi, wbv, outv, sem):
    wid = lax.axis_index("s") * NC + lax.axis_index("c")
    base = wid * BPW

    pltpu.sync_copy(x_hbm.at[pl.ds(base * 2, 2 * BPW)], xv)
    pltpu.sync_copy(wb_hbm, wbv)

    iota16 = lax.iota(jnp.int32, 16)

    for j in range(NGROUP):
        chunk, off = (j * 16) // 128, (j * 16) % 128
        ucol = plsc.load_gather(xv, [iota16 * 2 + (j * 32)])
        icol = plsc.load_gather(xv, [iota16 * 2 + (j * 32 + 1)])
        uidx_lo[chunk, pl.ds(off, 16)] = ucol * 2
        uidx_hi[chunk, pl.ds(off, 16)] = ucol * 2 + 1
        iidx_lo[chunk, pl.ds(off, 16)] = icol * 2
        iidx_hi[chunk, pl.ds(off, 16)] = icol * 2 + 1

    copies = []
    for c in range(NCHUNK):
        for idxref, table, dst in ((uidx_lo, user_hbm, ulo),
                                   (uidx_hi, user_hbm, uhi),
                                   (iidx_lo, item_hbm, ilo),
                                   (iidx_hi, item_hbm, ihi)):
            copies.append(pltpu.async_copy(
                table.at[idxref.at[c]], dst.at[pl.ds(c * 128, 128)], sem))
    for cp in copies:
        cp.wait()

    w0 = wbv[pl.ds(0, 16)]
    w1 = wbv[pl.ds(16, 16)]
    bias = wbv[pl.ds(24, 16)][8]

    def group(g, carry):
        rows = iota16 + g * 16
        acc = jnp.zeros((16,), jnp.float32)
        for k in range(K):
            uref = ulo if k < 16 else uhi
            iref = ilo if k < 16 else ihi
            kvec = jnp.full((16,), k % 16, jnp.int32)
            ucol = plsc.load_gather(uref, [rows, kvec])
            icol = plsc.load_gather(iref, [rows, kvec])
            wk = w0[k] if k < 16 else w1[k - 16]
            acc = acc + ucol * icol * wk
        z = acc + bias
        outv[pl.ds(g * 16, 16)] = 1.0 / (1.0 + jnp.exp(-z))
        return carry

    lax.fori_loop(0, NGROUP, group, None)

    pltpu.sync_copy(outv, out_hbm.at[pl.ds(base, BPW)])


@jax.jit
def kernel(x, user_table, item_table, W, b):
    wb = jnp.concatenate([W.reshape(K), jnp.pad(b, (0, 7))]).astype(jnp.float32)
    mesh = plsc.VectorSubcoreMesh(core_axis_name="c", subcore_axis_name="s")
    out = pl.kernel(
        _gmf_body,
        out_type=jax.ShapeDtypeStruct((B,), jnp.float32),
        mesh=mesh,
        compiler_params=pltpu.CompilerParams(
            needs_layout_passes=False, use_tc_tiling_on_sc=False),
        scratch_types=[
            pltpu.VMEM((2 * BPW,), jnp.int32),
            pltpu.VMEM((NCHUNK, 128), jnp.int32),
            pltpu.VMEM((NCHUNK, 128), jnp.int32),
            pltpu.VMEM((NCHUNK, 128), jnp.int32),
            pltpu.VMEM((NCHUNK, 128), jnp.int32),
            pltpu.VMEM((BPW, 16), jnp.float32),
            pltpu.VMEM((BPW, 16), jnp.float32),
            pltpu.VMEM((BPW, 16), jnp.float32),
            pltpu.VMEM((BPW, 16), jnp.float32),
            pltpu.VMEM((40,), jnp.float32),
            pltpu.VMEM((BPW,), jnp.float32),
            pltpu.SemaphoreType.DMA,
        ],
    )(x.astype(jnp.int32).reshape(2 * B),
      wb,
      user_table.reshape(-1, 16),
      item_table.reshape(-1, 16))
    return out.reshape(B, 1, 1)
